# baseline (speedup 1.0000x reference)
import functools

import jax
import jax.numpy as jnp
from jax import lax
from jax.experimental import pallas as pl
from jax.experimental.pallas import tpu as pltpu
from jax.experimental.pallas import tpu_sc as plsc

_GAMMA = 12.0
_HIDDEN = 64
_BATCH = 16384
_NW = 32
_BPW = _BATCH // _NW
_CHUNK = 128
_PAIRS = _CHUNK // 2
_NCHUNK = _BPW // _PAIRS
_TBLK = 8192
_TSH = 13
_NTBLK = (1000000 + _TBLK - 1) // _TBLK


def _sqrt16(x):
    xs = jnp.maximum(x, jnp.float32(1.1754944e-38))
    i = lax.bitcast_convert_type(xs, jnp.int32)
    i = jnp.int32(0x5F3759DF) - lax.shift_right_arithmetic(i, jnp.int32(1))
    y = lax.bitcast_convert_type(i, jnp.float32)
    for _ in range(3):
        y = y * (jnp.float32(1.5) - jnp.float32(0.5) * xs * y * y)
    return xs * y


def _make_sc_kernel():
    mesh = plsc.VectorSubcoreMesh(core_axis_name="c", subcore_axis_name="s")

    @functools.partial(
        pl.kernel,
        mesh=mesh,
        compiler_params=pltpu.CompilerParams(
            needs_layout_passes=False, use_tc_tiling_on_sc=True),
        out_type=jax.ShapeDtypeStruct((_NW, _BPW), jnp.float32),
        scratch_types=[
            pltpu.VMEM((_NCHUNK, _CHUNK), jnp.int32),
            pltpu.VMEM((_NCHUNK, _CHUNK), jnp.int32),
            pltpu.VMEM((_CHUNK, 128), jnp.int32),
            pltpu.VMEM((_CHUNK, 128), jnp.int32),
            pltpu.VMEM((_BPW,), jnp.float32),
            pltpu.SemaphoreType.DMA,
        ],
    )
    def sc_kernel(table_hbm, idx_hbm, out_hbm, idx_v, rowid_v, rows0_v,
                  rows1_v, scores_v, sem):
        wid = lax.axis_index("s") * 2 + lax.axis_index("c")

        pltpu.sync_copy(idx_hbm.at[wid], idx_v)

        for c in range(_NCHUNK):
            for k in range(_CHUNK // 16):
                v = idx_v[c, pl.ds(k * 16, 16)]
                rowid_v[c, pl.ds(k * 16, 16)] = (
                    lax.shift_left(lax.shift_right_logical(
                        v, jnp.int32(_TSH)), jnp.int32(_TSH - 2))
                    + (v & jnp.int32(_TBLK // 4 - 1)))

        def fire(c, buf):
            pltpu.async_copy(table_hbm.at[rowid_v.at[c]], buf, sem)

        def drain(buf):
            pltpu.make_async_copy(
                table_hbm.at[pl.ds(0, _CHUNK)], buf, sem).wait()

        iota16 = lax.iota(jnp.int32, 16)

        def compute(buf, c):
            cv = jnp.full((16,), 0, jnp.int32) + c

            def group(g, carry):
                pair = iota16 + g * 16
                hv = plsc.load_gather(idx_v, [cv, pair * 2])
                tv = plsc.load_gather(idx_v, [cv, pair * 2 + 1])
                hoff = (lax.shift_right_logical(hv, jnp.int32(_TSH - 2))
                        & jnp.int32(3)) * jnp.int32(32)
                toff = (lax.shift_right_logical(tv, jnp.int32(_TSH - 2))
                        & jnp.int32(3)) * jnp.int32(32)
                hrow = pair * 2
                trow = pair * 2 + 1
                accs = [jnp.zeros((16,), jnp.float32) for _ in range(4)]
                m16 = jnp.int32(-65536)
                for d in range(_HIDDEN // 2):
                    wh = plsc.load_gather(buf, [hrow, hoff + d])
                    wt = plsc.load_gather(buf, [trow, toff + d])
                    hl = lax.bitcast_convert_type(
                        lax.shift_left(wh, jnp.int32(16)), jnp.float32)
                    hh = lax.bitcast_convert_type(wh & m16, jnp.float32)
                    tl = lax.bitcast_convert_type(
                        lax.shift_left(wt, jnp.int32(16)), jnp.float32)
                    th = lax.bitcast_convert_type(wt & m16, jnp.float32)
                    dl = hl - tl
                    dh = hh - th
                    accs[d % 4] = accs[d % 4] + dl * dl
                    accs[(d + 2) % 4] = accs[(d + 2) % 4] + dh * dh
                total = (accs[0] + accs[1]) + (accs[2] + accs[3])
                scores_v[pl.ds(c * _PAIRS + g * 16, 16)] = (
                    jnp.float32(_GAMMA) - _sqrt16(total))
                return carry

            lax.fori_loop(0, _PAIRS // 16, group, 0, unroll=4)

        fire(0, rows0_v)
        fire(1, rows1_v)

        def pairbody(i, carry):
            c0 = 2 * i
            drain(rows0_v)
            compute(rows0_v, c0)
            fire(c0 + 2, rows0_v)
            drain(rows1_v)
            compute(rows1_v, c0 + 1)
            fire(c0 + 3, rows1_v)
            return carry

        lax.fori_loop(0, _NCHUNK // 2 - 1, pairbody, 0)

        drain(rows0_v)
        compute(rows0_v, _NCHUNK - 2)
        drain(rows1_v)
        compute(rows1_v, _NCHUNK - 1)

        pltpu.sync_copy(scores_v, out_hbm.at[wid])

    return sc_kernel


_sc_kernel = _make_sc_kernel()


def _transpose_body(in_ref, out_ref):
    x = in_ref[...]
    lo = lax.bitcast_convert_type(x[: _HIDDEN // 2, :], jnp.int32)
    hi = lax.bitcast_convert_type(x[_HIDDEN // 2:, :], jnp.int32)
    lo16 = lax.shift_right_logical(lo + jnp.int32(0x8000), jnp.int32(16))
    hi16 = (hi + jnp.int32(0x8000)) & jnp.int32(-65536)
    w = hi16 | lo16
    q = _TBLK // 4
    wq = jnp.concatenate(
        [w[:, :q], w[:, q:2 * q], w[:, 2 * q:3 * q], w[:, 3 * q:]], axis=0)
    out_ref[...] = jnp.transpose(wq)


_tc_transpose = pl.pallas_call(
    _transpose_body,
    grid=(_NTBLK,),
    in_specs=[pl.BlockSpec((_HIDDEN, _TBLK), lambda j: (0, j))],
    out_specs=pl.BlockSpec((_TBLK // 4, 128), lambda j: (j, 0)),
    out_shape=jax.ShapeDtypeStruct((_NTBLK * (_TBLK // 4), 128), jnp.int32),
)


@jax.jit
def kernel(sample, entity_embedding):
    table2 = _tc_transpose(entity_embedding.T)
    idx = sample.reshape(_NW, _NCHUNK, _CHUNK)
    out = _sc_kernel(table2, idx)
    return out.reshape(_BATCH, 1)


# speedup vs baseline: 1.1715x; 1.1715x over previous
import functools

import jax
import jax.numpy as jnp
from jax import lax
from jax.experimental import pallas as pl
from jax.experimental.pallas import tpu as pltpu
from jax.experimental.pallas import tpu_sc as plsc

_GAMMA = 12.0
_HIDDEN = 64
_BATCH = 16384
_NW = 32
_BPW = _BATCH // _NW
_CHUNK = 128
_PAIRS = _CHUNK // 2
_NCHUNK = _BPW // _PAIRS
_TBLK = 16384
_TSH = 14
_NTBLK = (1000000 + _TBLK - 1) // _TBLK


def _sqrt16(x):
    xs = jnp.maximum(x, jnp.float32(1.1754944e-38))
    i = lax.bitcast_convert_type(xs, jnp.int32)
    i = jnp.int32(0x5F3759DF) - lax.shift_right_arithmetic(i, jnp.int32(1))
    y = lax.bitcast_convert_type(i, jnp.float32)
    for _ in range(3):
        y = y * (jnp.float32(1.5) - jnp.float32(0.5) * xs * y * y)
    return xs * y


def _make_sc_kernel():
    mesh = plsc.VectorSubcoreMesh(core_axis_name="c", subcore_axis_name="s")

    @functools.partial(
        pl.kernel,
        mesh=mesh,
        compiler_params=pltpu.CompilerParams(
            needs_layout_passes=False, use_tc_tiling_on_sc=True),
        out_type=jax.ShapeDtypeStruct((_NW, _BPW), jnp.float32),
        scratch_types=[
            pltpu.VMEM((_NCHUNK, _CHUNK), jnp.int32),
            pltpu.VMEM((_NCHUNK, _CHUNK), jnp.int32),
            pltpu.VMEM((_CHUNK, 128), jnp.int32),
            pltpu.VMEM((_CHUNK, 128), jnp.int32),
            pltpu.VMEM((_BPW,), jnp.float32),
            pltpu.SemaphoreType.DMA,
        ],
    )
    def sc_kernel(table_hbm, idx_hbm, out_hbm, idx_v, rowid_v, rows0_v,
                  rows1_v, scores_v, sem):
        wid = lax.axis_index("s") * 2 + lax.axis_index("c")

        pltpu.sync_copy(idx_hbm.at[wid], idx_v)

        for c in range(_NCHUNK):
            for k in range(_CHUNK // 16):
                v = idx_v[c, pl.ds(k * 16, 16)]
                rowid_v[c, pl.ds(k * 16, 16)] = (
                    lax.shift_left(lax.shift_right_logical(
                        v, jnp.int32(_TSH)), jnp.int32(_TSH - 2))
                    + (v & jnp.int32(_TBLK // 4 - 1)))

        def fire(c, buf):
            pltpu.async_copy(table_hbm.at[rowid_v.at[c]], buf, sem)

        def drain(buf):
            pltpu.make_async_copy(
                table_hbm.at[pl.ds(0, _CHUNK)], buf, sem).wait()

        iota16 = lax.iota(jnp.int32, 16)

        def compute(buf, c):
            cv = jnp.full((16,), 0, jnp.int32) + c

            def group(g, carry):
                pair = iota16 + g * 16
                hv = plsc.load_gather(idx_v, [cv, pair * 2])
                tv = plsc.load_gather(idx_v, [cv, pair * 2 + 1])
                hoff = (lax.shift_right_logical(hv, jnp.int32(_TSH - 2))
                        & jnp.int32(3)) * jnp.int32(32)
                toff = (lax.shift_right_logical(tv, jnp.int32(_TSH - 2))
                        & jnp.int32(3)) * jnp.int32(32)
                hrow = pair * 2
                trow = pair * 2 + 1
                accs = [jnp.zeros((16,), jnp.float32) for _ in range(4)]
                m16 = jnp.int32(-65536)
                for d in range(_HIDDEN // 2):
                    wh = plsc.load_gather(buf, [hrow, hoff + d])
                    wt = plsc.load_gather(buf, [trow, toff + d])
                    hl = lax.bitcast_convert_type(
                        lax.shift_left(wh, jnp.int32(16)), jnp.float32)
                    hh = lax.bitcast_convert_type(wh & m16, jnp.float32)
                    tl = lax.bitcast_convert_type(
                        lax.shift_left(wt, jnp.int32(16)), jnp.float32)
                    th = lax.bitcast_convert_type(wt & m16, jnp.float32)
                    dl = hl - tl
                    dh = hh - th
                    accs[d % 4] = accs[d % 4] + dl * dl
                    accs[(d + 2) % 4] = accs[(d + 2) % 4] + dh * dh
                total = (accs[0] + accs[1]) + (accs[2] + accs[3])
                scores_v[pl.ds(c * _PAIRS + g * 16, 16)] = (
                    jnp.float32(_GAMMA) - _sqrt16(total))
                return carry

            lax.fori_loop(0, _PAIRS // 16, group, 0, unroll=4)

        fire(0, rows0_v)
        fire(1, rows1_v)

        def pairbody(i, carry):
            c0 = 2 * i
            drain(rows0_v)
            compute(rows0_v, c0)
            fire(c0 + 2, rows0_v)
            drain(rows1_v)
            compute(rows1_v, c0 + 1)
            fire(c0 + 3, rows1_v)
            return carry

        lax.fori_loop(0, _NCHUNK // 2 - 1, pairbody, 0)

        drain(rows0_v)
        compute(rows0_v, _NCHUNK - 2)
        drain(rows1_v)
        compute(rows1_v, _NCHUNK - 1)

        pltpu.sync_copy(scores_v, out_hbm.at[wid])

    return sc_kernel


_sc_kernel = _make_sc_kernel()


def _transpose_body(in_ref, out_ref):
    x = in_ref[...]
    lo = lax.bitcast_convert_type(x[: _HIDDEN // 2, :], jnp.int32)
    hi = lax.bitcast_convert_type(x[_HIDDEN // 2:, :], jnp.int32)
    lo16 = lax.shift_right_logical(lo + jnp.int32(0x8000), jnp.int32(16))
    hi16 = (hi + jnp.int32(0x8000)) & jnp.int32(-65536)
    w = hi16 | lo16
    q = _TBLK // 4
    wq = jnp.concatenate(
        [w[:, :q], w[:, q:2 * q], w[:, 2 * q:3 * q], w[:, 3 * q:]], axis=0)
    out_ref[...] = jnp.transpose(wq)


_tc_transpose = pl.pallas_call(
    _transpose_body,
    grid=(_NTBLK,),
    in_specs=[pl.BlockSpec((_HIDDEN, _TBLK), lambda j: (0, j))],
    out_specs=pl.BlockSpec((_TBLK // 4, 128), lambda j: (j, 0)),
    out_shape=jax.ShapeDtypeStruct((_NTBLK * (_TBLK // 4), 128), jnp.int32),
)


@jax.jit
def kernel(sample, entity_embedding):
    table2 = _tc_transpose(entity_embedding.T)
    idx = sample.reshape(_NW, _NCHUNK, _CHUNK)
    out = _sc_kernel(table2, idx)
    return out.reshape(_BATCH, 1)


# speedup vs baseline: 1.2512x; 1.0680x over previous
import functools

import jax
import jax.numpy as jnp
from jax import lax
from jax.experimental import pallas as pl
from jax.experimental.pallas import tpu as pltpu
from jax.experimental.pallas import tpu_sc as plsc

_GAMMA = 12.0
_HIDDEN = 64
_BATCH = 16384
_NW = 32
_BPW = _BATCH // _NW
_CHUNK = 128
_PAIRS = _CHUNK // 2
_NCHUNK = _BPW // _PAIRS
_TBLK = 32768
_TSH = 15
_NTBLK = (1000000 + _TBLK - 1) // _TBLK


def _sqrt16(x):
    xs = jnp.maximum(x, jnp.float32(1.1754944e-38))
    i = lax.bitcast_convert_type(xs, jnp.int32)
    i = jnp.int32(0x5F3759DF) - lax.shift_right_arithmetic(i, jnp.int32(1))
    y = lax.bitcast_convert_type(i, jnp.float32)
    for _ in range(3):
        y = y * (jnp.float32(1.5) - jnp.float32(0.5) * xs * y * y)
    return xs * y


def _make_sc_kernel():
    mesh = plsc.VectorSubcoreMesh(core_axis_name="c", subcore_axis_name="s")

    @functools.partial(
        pl.kernel,
        mesh=mesh,
        compiler_params=pltpu.CompilerParams(
            needs_layout_passes=False, use_tc_tiling_on_sc=True),
        out_type=jax.ShapeDtypeStruct((_NW, _BPW), jnp.float32),
        scratch_types=[
            pltpu.VMEM((_NCHUNK, _CHUNK), jnp.int32),
            pltpu.VMEM((_NCHUNK, _CHUNK), jnp.int32),
            pltpu.VMEM((_CHUNK, 128), jnp.int32),
            pltpu.VMEM((_CHUNK, 128), jnp.int32),
            pltpu.VMEM((_BPW,), jnp.float32),
            pltpu.SemaphoreType.DMA,
        ],
    )
    def sc_kernel(table_hbm, idx_hbm, out_hbm, idx_v, rowid_v, rows0_v,
                  rows1_v, scores_v, sem):
        wid = lax.axis_index("s") * 2 + lax.axis_index("c")

        pltpu.sync_copy(idx_hbm.at[wid], idx_v)

        for c in range(_NCHUNK):
            for k in range(_CHUNK // 16):
                v = idx_v[c, pl.ds(k * 16, 16)]
                rowid_v[c, pl.ds(k * 16, 16)] = (
                    lax.shift_left(lax.shift_right_logical(
                        v, jnp.int32(_TSH)), jnp.int32(_TSH - 2))
                    + (v & jnp.int32(_TBLK // 4 - 1)))

        def fire(c, buf):
            pltpu.async_copy(table_hbm.at[rowid_v.at[c]], buf, sem)

        def drain(buf):
            pltpu.make_async_copy(
                table_hbm.at[pl.ds(0, _CHUNK)], buf, sem).wait()

        iota16 = lax.iota(jnp.int32, 16)

        def compute(buf, c):
            cv = jnp.full((16,), 0, jnp.int32) + c

            def group(g, carry):
                pair = iota16 + g * 16
                hv = plsc.load_gather(idx_v, [cv, pair * 2])
                tv = plsc.load_gather(idx_v, [cv, pair * 2 + 1])
                hoff = (lax.shift_right_logical(hv, jnp.int32(_TSH - 2))
                        & jnp.int32(3)) * jnp.int32(32)
                toff = (lax.shift_right_logical(tv, jnp.int32(_TSH - 2))
                        & jnp.int32(3)) * jnp.int32(32)
                hrow = pair * 2
                trow = pair * 2 + 1
                accs = [jnp.zeros((16,), jnp.float32) for _ in range(4)]
                m16 = jnp.int32(-65536)
                for d in range(_HIDDEN // 2):
                    wh = plsc.load_gather(buf, [hrow, hoff + d])
                    wt = plsc.load_gather(buf, [trow, toff + d])
                    hl = lax.bitcast_convert_type(
                        lax.shift_left(wh, jnp.int32(16)), jnp.float32)
                    hh = lax.bitcast_convert_type(wh & m16, jnp.float32)
                    tl = lax.bitcast_convert_type(
                        lax.shift_left(wt, jnp.int32(16)), jnp.float32)
                    th = lax.bitcast_convert_type(wt & m16, jnp.float32)
                    dl = hl - tl
                    dh = hh - th
                    accs[d % 4] = accs[d % 4] + dl * dl
                    accs[(d + 2) % 4] = accs[(d + 2) % 4] + dh * dh
                total = (accs[0] + accs[1]) + (accs[2] + accs[3])
                scores_v[pl.ds(c * _PAIRS + g * 16, 16)] = (
                    jnp.float32(_GAMMA) - _sqrt16(total))
                return carry

            lax.fori_loop(0, _PAIRS // 16, group, 0, unroll=2)

        fire(0, rows0_v)
        fire(1, rows1_v)

        def pairbody(i, carry):
            c0 = 2 * i
            drain(rows0_v)
            compute(rows0_v, c0)
            fire(c0 + 2, rows0_v)
            drain(rows1_v)
            compute(rows1_v, c0 + 1)
            fire(c0 + 3, rows1_v)
            return carry

        lax.fori_loop(0, _NCHUNK // 2 - 1, pairbody, 0)

        drain(rows0_v)
        compute(rows0_v, _NCHUNK - 2)
        drain(rows1_v)
        compute(rows1_v, _NCHUNK - 1)

        pltpu.sync_copy(scores_v, out_hbm.at[wid])

    return sc_kernel


_sc_kernel = _make_sc_kernel()


def _transpose_body(in_ref, out_ref):
    x = in_ref[...]
    lo = lax.bitcast_convert_type(x[: _HIDDEN // 2, :], jnp.int32)
    hi = lax.bitcast_convert_type(x[_HIDDEN // 2:, :], jnp.int32)
    lo16 = lax.shift_right_logical(lo + jnp.int32(0x8000), jnp.int32(16))
    hi16 = (hi + jnp.int32(0x8000)) & jnp.int32(-65536)
    w = hi16 | lo16
    q = _TBLK // 4
    wq = jnp.concatenate(
        [w[:, :q], w[:, q:2 * q], w[:, 2 * q:3 * q], w[:, 3 * q:]], axis=0)
    out_ref[...] = jnp.transpose(wq)


_tc_transpose = pl.pallas_call(
    _transpose_body,
    grid=(_NTBLK,),
    in_specs=[pl.BlockSpec((_HIDDEN, _TBLK), lambda j: (0, j))],
    out_specs=pl.BlockSpec((_TBLK // 4, 128), lambda j: (j, 0)),
    out_shape=jax.ShapeDtypeStruct((_NTBLK * (_TBLK // 4), 128), jnp.int32),
)


@jax.jit
def kernel(sample, entity_embedding):
    table2 = _tc_transpose(entity_embedding.T)
    idx = sample.reshape(_NW, _NCHUNK, _CHUNK)
    out = _sc_kernel(table2, idx)
    return out.reshape(_BATCH, 1)
